# baseline (speedup 1.0000x reference)
import jax
import jax.numpy as jnp
from jax import lax
from jax.experimental import pallas as pl
from jax.experimental.pallas import tpu as pltpu
from jax.experimental.pallas import tpu_sc as plsc

EMBED_DIM = 32
MLP_HIDDEN = 64
BATCH = 16384

NC = 2
NS = 16
NW = NC * NS
BPW = BATCH // NW


def _sc_gather_body(ui_hbm, ii_hbm, uemb_hbm, iemb_hbm, urows_hbm, irows_hbm,
                    idx_u, idx_i, rows_u, rows_i, sem_u, sem_i):
    wid = lax.axis_index("s") * NC + lax.axis_index("c")
    base = wid * BPW
    pltpu.sync_copy(ui_hbm.at[pl.ds(base, BPW)], idx_u)
    pltpu.sync_copy(ii_hbm.at[pl.ds(base, BPW)], idx_i)
    cu = pltpu.async_copy(uemb_hbm.at[idx_u], rows_u, sem_u)
    ci = pltpu.async_copy(iemb_hbm.at[idx_i], rows_i, sem_i)
    cu.wait()
    ci.wait()
    pltpu.sync_copy(rows_u, urows_hbm.at[pl.ds(base, BPW)])
    pltpu.sync_copy(rows_i, irows_hbm.at[pl.ds(base, BPW)])


def _mlp_body(u_ref, v_ref, w1t_ref, b1_ref, w2_ref, b2_ref, out_ref):
    u = u_ref[...]
    v = v_ref[...]
    h = (jnp.dot(u, w1t_ref[:EMBED_DIM, :], preferred_element_type=jnp.float32)
         + jnp.dot(v, w1t_ref[EMBED_DIM:, :], preferred_element_type=jnp.float32)
         + b1_ref[...])
    h = jnp.maximum(h, 0.0)
    o = jnp.sum(h * w2_ref[...], axis=1) + b2_ref[0, 0]
    out_ref[...] = jax.nn.sigmoid(o)


def kernel(user_indices, item_indices, user_emb, item_emb, W1, b1, W2, b2):
    mesh = plsc.VectorSubcoreMesh(core_axis_name="c", subcore_axis_name="s")
    gather = pl.kernel(
        _sc_gather_body,
        mesh=mesh,
        compiler_params=pltpu.CompilerParams(use_tc_tiling_on_sc=False),
        out_type=[
            jax.ShapeDtypeStruct((BATCH, EMBED_DIM), jnp.float32),
            jax.ShapeDtypeStruct((BATCH, EMBED_DIM), jnp.float32),
        ],
        scratch_types=[
            pltpu.VMEM((BPW,), jnp.int32),
            pltpu.VMEM((BPW,), jnp.int32),
            pltpu.VMEM((BPW, EMBED_DIM), jnp.float32),
            pltpu.VMEM((BPW, EMBED_DIM), jnp.float32),
            pltpu.SemaphoreType.DMA,
            pltpu.SemaphoreType.DMA,
        ],
    )
    u_rows, v_rows = gather(user_indices.astype(jnp.int32),
                            item_indices.astype(jnp.int32),
                            user_emb, item_emb)

    out = pl.pallas_call(
        _mlp_body,
        out_shape=jax.ShapeDtypeStruct((BATCH,), jnp.float32),
    )(u_rows, v_rows, W1.T, b1.reshape(1, MLP_HIDDEN), W2, b2.reshape(1, 1))
    return out
